# baseline (speedup 1.0000x reference)
import functools

import jax
import jax.numpy as jnp
from jax import lax
from jax.experimental import pallas as pl
from jax.experimental.pallas import tpu as pltpu
from jax.experimental.pallas import tpu_sc as plsc

_N = 10000
_E = 160000
_F = 16
_NOUT = 40

_NC = 2
_NS = 16
_NW = _NC * _NS

_SUB = 128
_STG = 1024
_NSUB = _STG // _SUB
_NSTG = 5
_EPW = _STG * _NSTG
_E_PAD = _EPW * _NW

_STRIPE = 640
_N_PAD = _STRIPE * _NS
_TRASH = _N

_i32 = jnp.int32


def _iota16():
    return lax.iota(_i32, 16)


def _sc_gather_body(x_hbm, src2_hbm, xjT_hbm, idx2, rows3, sem_i, sem_g, sem_o):
    c = lax.axis_index("c")
    s = lax.axis_index("s")
    wid = s * _i32(_NC) + c
    base = wid * _i32(_EPW)
    base_row = wid * _i32(_EPW // _SUB)

    def load_idx(g):
        return pltpu.async_copy(
            src2_hbm.at[pl.ds(base_row + _i32(g * _NSUB), _NSUB)],
            idx2.at[_i32(g & 1)], sem_i)

    def issue_gathers(g):
        b = g % 3
        return [pltpu.async_copy(
            x_hbm.at[idx2.at[_i32(g & 1), _i32(j)]],
            rows3.at[_i32(b)].at[pl.ds(_i32(j * _SUB), _SUB)], sem_g)
            for j in range(_NSUB)]

    idx_d = load_idx(0)
    idx_d.wait()
    gd = {0: issue_gathers(0)}
    if _NSTG > 1:
        idx_d = load_idx(1)
    out_d = [None, None, None]
    for g in range(_NSTG):
        b = g % 3
        for d in gd.pop(g):
            d.wait()
        if g + 1 < _NSTG:
            idx_d.wait()
            nb = (g + 1) % 3
            if out_d[nb] is not None:
                for d in out_d[nb]:
                    d.wait()
                out_d[nb] = None
            gd[g + 1] = issue_gathers(g + 1)
            if g + 2 < _NSTG:
                idx_d = load_idx(g + 2)
        off = base + _i32(g * _STG)
        out_d[b] = [pltpu.async_copy(
            rows3.at[_i32(b)].at[:, pl.ds(_i32(f), 1)],
            xjT_hbm.at[_i32(f)].at[pl.ds(off, _STG), pl.ds(_i32(0), 1)],
            sem_o)
            for f in range(_F)]
    for ds_ in out_d:
        if ds_ is not None:
            for d in ds_:
                d.wait()


def _sc_scatter_body(msgT_hbm, dst2_hbm, aggT_hbm, deg_hbm, idx2, rows2,
                     deg_v, zbuf_v, aggT_v, agg_sh, sem_i, sem_m):
    c = lax.axis_index("c")
    s = lax.axis_index("s")
    wid = s * _i32(_NC) + c
    base = wid * _i32(_EPW)
    base_row = wid * _i32(_EPW // _SUB)
    z16 = jnp.zeros((_F,), jnp.float32)

    def load_idx(g):
        return pltpu.async_copy(
            dst2_hbm.at[pl.ds(base_row + _i32(g * _NSUB), _NSUB)],
            idx2.at[_i32(g & 1)], sem_i)

    def load_msg(g):
        off = base + _i32(g * _STG)
        return [pltpu.async_copy(
            msgT_hbm.at[_i32(f)].at[pl.ds(off, _STG), pl.ds(_i32(0), 1)],
            rows2.at[_i32(g & 1)].at[:, pl.ds(_i32(f), 1)], sem_m)
            for f in range(_F)]

    idx_d = load_idx(0)
    msg_d = load_msg(0)

    def zrow(i, carry):
        zbuf_v[i, :] = z16
        return carry

    lax.fori_loop(_i32(0), _i32(_STRIPE), zrow, _i32(0))

    def zdeg(i, carry):
        deg_v[pl.ds(i * _i32(_F), _F)] = z16
        return carry

    lax.fori_loop(_i32(0), _i32(_N_PAD // _F), zdeg, _i32(0))

    pltpu.sync_copy(zbuf_v, agg_sh.at[pl.ds(s * _i32(_STRIPE), _STRIPE)])
    plsc.subcore_barrier()

    ones16 = jnp.ones((_F,), jnp.float32)
    for g in range(_NSTG):
        b = g & 1
        idx_d.wait()
        for d in msg_d:
            d.wait()
        if g + 1 < _NSTG:
            idx_d = load_idx(g + 1)
            msg_d = load_msg(g + 1)
        for j in range(_NSUB):
            for i in range(_SUB // _F):
                iv = idx2[_i32(b), _i32(j), pl.ds(_i32(i * _F), _F)]
                plsc.addupdate_scatter(deg_v, [iv], ones16)
        for j in range(_NSUB):
            pltpu.sync_copy(
                rows2.at[_i32(b)].at[pl.ds(_i32(j * _SUB), _SUB)],
                agg_sh.at[idx2.at[_i32(b), _i32(j)]], add=True)
    plsc.subcore_barrier()

    pltpu.sync_copy(agg_sh.at[pl.ds(s * _i32(_STRIPE), _STRIPE)], zbuf_v)

    def trs_body(l8, carry):
        ridx = l8 * _i32(16) + _iota16()
        for f in range(_F):
            cidx = jnp.full((16,), f, _i32)
            v = plsc.load_gather(zbuf_v, [ridx, cidx])
            aggT_v[_i32(f), pl.ds(l8 * _i32(16), 16)] = v
        return carry

    lax.fori_loop(_i32(0), _i32(_STRIPE // 16), trs_body, _i32(0))
    pltpu.sync_copy(aggT_v,
                    aggT_hbm.at[c].at[:, pl.ds(s * _i32(_STRIPE), _STRIPE)])
    pltpu.sync_copy(deg_v, deg_hbm.at[wid])


@functools.cache
def _sc_kernels():
    mesh = plsc.VectorSubcoreMesh(core_axis_name="c", subcore_axis_name="s",
                                  num_cores=_NC, num_subcores=_NS)
    params = pltpu.CompilerParams(use_tc_tiling_on_sc=False,
                                  needs_layout_passes=False)
    gather = pl.kernel(
        _sc_gather_body,
        out_type=jax.ShapeDtypeStruct((_F, _E_PAD, 1), jnp.float32),
        mesh=mesh,
        compiler_params=params,
        scratch_types=[
            pltpu.VMEM((2, _NSUB, _SUB), jnp.int32),
            pltpu.VMEM((3, _STG, _F), jnp.float32),
            pltpu.SemaphoreType.DMA,
            pltpu.SemaphoreType.DMA,
            pltpu.SemaphoreType.DMA,
        ],
    )
    scatter = pl.kernel(
        _sc_scatter_body,
        out_type=[
            jax.ShapeDtypeStruct((_NC, _F, _N_PAD), jnp.float32),
            jax.ShapeDtypeStruct((_NW, _N_PAD), jnp.float32),
        ],
        mesh=mesh,
        compiler_params=params,
        scratch_types=[
            pltpu.VMEM((2, _NSUB, _SUB), jnp.int32),
            pltpu.VMEM((2, _STG, _F), jnp.float32),
            pltpu.VMEM((_N_PAD,), jnp.float32),
            pltpu.VMEM((_STRIPE, _F), jnp.float32),
            pltpu.VMEM((_F, _STRIPE), jnp.float32),
            pltpu.VMEM_SHARED((_N_PAD, _F), jnp.float32),
            pltpu.SemaphoreType.DMA,
            pltpu.SemaphoreType.DMA,
        ],
    )
    return gather, scatter


_BLK = 16384


def _tc_msg_body(ea_ref, xj_ref, w1t_ref, b1c_ref, w2t_ref, b2c_ref,
                 w3pt_ref, b3pt_ref, msg_ref):
    f32 = jnp.float32
    h = jnp.dot(w1t_ref[...], ea_ref[...], preferred_element_type=f32)
    h = jnp.maximum(h + b1c_ref[...], 0.0)
    h = jnp.dot(w2t_ref[...], h, preferred_element_type=f32)
    h = jnp.maximum(h + b2c_ref[...], 0.0)
    tT = jnp.dot(w3pt_ref[...], h, preferred_element_type=f32) + b3pt_ref[...]
    xj = xj_ref[...]
    acc = tT[0:_F, :] * xj[0:1, :]
    for i in range(1, _F):
        acc = acc + tT[i * _F:(i + 1) * _F, :] * xj[i:i + 1, :]
    msg_ref[...] = acc


def _tc_msg(eaT, xjT, w1t, b1c, w2t, b2c, w3pt, b3pt):
    grid = _E_PAD // _BLK
    blk = lambda i: (jnp.int32(0), i)
    fixed = lambda i: (jnp.int32(0), jnp.int32(0))
    full = lambda shape: pl.BlockSpec(shape, fixed)
    return pl.pallas_call(
        _tc_msg_body,
        grid=(grid,),
        in_specs=[
            pl.BlockSpec((4, _BLK), blk),
            pl.BlockSpec((_F, _BLK), blk),
            full((_F, 4)), full((_F, 1)), full((32, _F)), full((32, 1)),
            full((256, 32)), full((256, 1)),
        ],
        out_specs=pl.BlockSpec((_F, _BLK), blk),
        out_shape=jax.ShapeDtypeStruct((_F, _E_PAD), jnp.float32),
        compiler_params=pltpu.CompilerParams(
            dimension_semantics=("arbitrary",)),
    )(eaT, xjT, w1t, b1c, w2t, b2c, w3pt, b3pt)


def _tc_final_body(agg_ref, deg_ref, gamma_ref, beta_ref, wf_ref, bf_ref,
                   out_ref):
    agg = agg_ref[0, :, :] + agg_ref[1, :, :]
    deg = jnp.sum(deg_ref[...], axis=0, keepdims=True)
    deg = jnp.maximum(deg, 1.0)
    out = agg / deg
    cid = lax.broadcasted_iota(jnp.int32, (_F, _N_PAD), 1)
    valid = cid < _N
    outm = jnp.where(valid, out, 0.0)
    inv_n = 1.0 / _N
    mu = jnp.sum(outm, axis=1, keepdims=True) * inv_n
    ex2 = jnp.sum(outm * outm, axis=1, keepdims=True) * inv_n
    var = ex2 - mu * mu
    scale = lax.rsqrt(var + 1e-5) * gamma_ref[...]
    out = (out - mu) * scale + beta_ref[...]
    out = jnp.maximum(out, 0.0)
    out_ref[...] = lax.dot_general(
        wf_ref[...], out, (((0,), (0,)), ((), ())),
        preferred_element_type=jnp.float32) + bf_ref[...]


def _tc_final(aggT, deg32, gamma, beta, wf, bf):
    return pl.pallas_call(
        _tc_final_body,
        out_shape=jax.ShapeDtypeStruct((_NOUT, _N_PAD), jnp.float32),
    )(aggT, deg32, gamma, beta, wf, bf)


def kernel(x, edge_index, edge_attr, W1, b1, W2, b2, W3, b3, gamma, beta,
           Wf, bf):
    f32 = jnp.float32
    x = x.astype(f32)
    src = edge_index[0].astype(jnp.int32)
    dst = edge_index[1].astype(jnp.int32)
    npad = _E_PAD - _E
    src = jnp.concatenate([src, jnp.zeros((npad,), jnp.int32)])
    dst = jnp.concatenate([dst, jnp.full((npad,), _TRASH, jnp.int32)])
    eaT = jnp.concatenate(
        [edge_attr.astype(f32).T, jnp.zeros((4, npad), f32)], axis=1)

    W3pt = W3.astype(f32).reshape(32, _F, _F).transpose(2, 1, 0).reshape(256, 32)
    b3pt = b3.astype(f32).reshape(_F, _F).T.reshape(256, 1)

    src2 = src.reshape(_E_PAD // _SUB, _SUB)
    dst2 = dst.reshape(_E_PAD // _SUB, _SUB)
    sc_gather, sc_scatter = _sc_kernels()
    xjT = sc_gather(x, src2).reshape(_F, _E_PAD)
    msgT = _tc_msg(eaT, xjT, W1.astype(f32).T, b1.astype(f32).reshape(_F, 1),
                   W2.astype(f32).T, b2.astype(f32).reshape(32, 1),
                   W3pt, b3pt)
    aggT, deg32 = sc_scatter(msgT.reshape(_F, _E_PAD, 1), dst2)
    outT = _tc_final(aggT, deg32, gamma.astype(f32).reshape(_F, 1),
                     beta.astype(f32).reshape(_F, 1), Wf.astype(f32),
                     bf.astype(f32).reshape(_NOUT, 1))
    return outT[:, :_N].T.astype(jnp.float64)


# speedup vs baseline: 55.6679x; 55.6679x over previous
import functools

import jax
import jax.numpy as jnp
from jax import lax
from jax.experimental import pallas as pl
from jax.experimental.pallas import tpu as pltpu
from jax.experimental.pallas import tpu_sc as plsc

_N = 10000
_E = 160000
_F = 16
_NOUT = 40

_NC = 2
_NS = 16
_NW = _NC * _NS

_SUB = 128
_STG = 1024
_NSUB = _STG // _SUB
_NSTG = 5
_EPW = _STG * _NSTG
_E_PAD = _EPW * _NW

_STRIPE = 640
_N_PAD = _STRIPE * _NS
_TRASH = _N

_i32 = jnp.int32


def _iota16():
    return lax.iota(_i32, 16)


def _sc_gather_body(x_hbm, src2_hbm, xjT_hbm, idx2, rows2, strip2,
                    sem_i, sem_g, sem_o):
    c = lax.axis_index("c")
    s = lax.axis_index("s")
    wid = s * _i32(_NC) + c
    base = wid * _i32(_EPW)
    base_row = wid * _i32(_EPW // _SUB)

    def load_idx(g):
        return pltpu.async_copy(
            src2_hbm.at[pl.ds(base_row + _i32(g * _NSUB), _NSUB)],
            idx2.at[_i32(g & 1)], sem_i)

    def issue_gathers(g):
        b = g & 1
        return [pltpu.async_copy(
            x_hbm.at[idx2.at[_i32(b), _i32(j)]],
            rows2.at[_i32(b)].at[pl.ds(_i32(j * _SUB), _SUB)], sem_g)
            for j in range(_NSUB)]

    idx_d = load_idx(0)
    idx_d.wait()
    gd = {0: issue_gathers(0)}
    if _NSTG > 1:
        idx_d = load_idx(1)
    out_d = [None, None]
    for g in range(_NSTG):
        b = g & 1
        for d in gd.pop(g):
            d.wait()
        if g + 1 < _NSTG:
            idx_d.wait()
            gd[g + 1] = issue_gathers(g + 1)
            if g + 2 < _NSTG:
                idx_d = load_idx(g + 2)
        if out_d[b] is not None:
            for d in out_d[b]:
                d.wait()
        rows_b = rows2.at[_i32(b)]

        def tr_body(l8, carry):
            ridx = l8 * _i32(16) + _iota16()
            for f in range(_F):
                cidx = jnp.full((16,), f, _i32)
                v = plsc.load_gather(rows_b, [ridx, cidx])
                strip2[_i32(b), _i32(f), pl.ds(l8 * _i32(16), 16)] = v
            return carry

        lax.fori_loop(_i32(0), _i32(_STG // 16), tr_body, _i32(0))
        off = base + _i32(g * _STG)
        out_d[b] = [pltpu.async_copy(
            strip2.at[_i32(b), _i32(f)],
            xjT_hbm.at[_i32(f)].at[pl.ds(off, _STG)], sem_o)
            for f in range(_F)]
    for ds_ in out_d:
        if ds_ is not None:
            for d in ds_:
                d.wait()


def _sc_scatter_body(msgT_hbm, dst2_hbm, aggT_hbm, deg_hbm, idx2, rows2,
                     strip2, deg_v, zbuf_v, aggT_v, agg_sh, sem_i, sem_m):
    c = lax.axis_index("c")
    s = lax.axis_index("s")
    wid = s * _i32(_NC) + c
    base = wid * _i32(_EPW)
    base_row = wid * _i32(_EPW // _SUB)
    z16 = jnp.zeros((_F,), jnp.float32)

    def load_idx(g):
        return pltpu.async_copy(
            dst2_hbm.at[pl.ds(base_row + _i32(g * _NSUB), _NSUB)],
            idx2.at[_i32(g & 1)], sem_i)

    def load_msg(g):
        off = base + _i32(g * _STG)
        return [pltpu.async_copy(
            msgT_hbm.at[_i32(f)].at[pl.ds(off, _STG)],
            strip2.at[_i32(g & 1), _i32(f)], sem_m)
            for f in range(_F)]

    idx_d = load_idx(0)
    msg_d = load_msg(0)

    def zrow(i, carry):
        zbuf_v[i, :] = z16
        return carry

    lax.fori_loop(_i32(0), _i32(_STRIPE), zrow, _i32(0))

    def zdeg(i, carry):
        deg_v[pl.ds(i * _i32(_F), _F)] = z16
        return carry

    lax.fori_loop(_i32(0), _i32(_N_PAD // _F), zdeg, _i32(0))

    pltpu.sync_copy(zbuf_v, agg_sh.at[pl.ds(s * _i32(_STRIPE), _STRIPE)])
    plsc.subcore_barrier()

    ones16 = jnp.ones((_F,), jnp.float32)
    for g in range(_NSTG):
        b = g & 1
        idx_d.wait()
        for d in msg_d:
            d.wait()
        rows_b = rows2.at[_i32(b)]

        def tr_body(l8, carry):
            ridx = l8 * _i32(16) + _iota16()
            for f in range(_F):
                cidx = jnp.full((16,), f, _i32)
                v = strip2[_i32(b), _i32(f), pl.ds(l8 * _i32(16), 16)]
                plsc.store_scatter(rows_b, [ridx, cidx], v)
            return carry

        lax.fori_loop(_i32(0), _i32(_STG // 16), tr_body, _i32(0))
        if g + 1 < _NSTG:
            idx_d = load_idx(g + 1)
            msg_d = load_msg(g + 1)
        for j in range(_NSUB):
            for i in range(_SUB // _F):
                iv = idx2[_i32(b), _i32(j), pl.ds(_i32(i * _F), _F)]
                plsc.addupdate_scatter(deg_v, [iv], ones16)
        for j in range(_NSUB):
            pltpu.sync_copy(
                rows2.at[_i32(b)].at[pl.ds(_i32(j * _SUB), _SUB)],
                agg_sh.at[idx2.at[_i32(b), _i32(j)]], add=True)
    plsc.subcore_barrier()

    pltpu.sync_copy(agg_sh.at[pl.ds(s * _i32(_STRIPE), _STRIPE)], zbuf_v)

    def trs_body(l8, carry):
        ridx = l8 * _i32(16) + _iota16()
        for f in range(_F):
            cidx = jnp.full((16,), f, _i32)
            v = plsc.load_gather(zbuf_v, [ridx, cidx])
            aggT_v[_i32(f), pl.ds(l8 * _i32(16), 16)] = v
        return carry

    lax.fori_loop(_i32(0), _i32(_STRIPE // 16), trs_body, _i32(0))
    pltpu.sync_copy(aggT_v,
                    aggT_hbm.at[c].at[:, pl.ds(s * _i32(_STRIPE), _STRIPE)])
    pltpu.sync_copy(deg_v, deg_hbm.at[wid])


@functools.cache
def _sc_kernels():
    mesh = plsc.VectorSubcoreMesh(core_axis_name="c", subcore_axis_name="s",
                                  num_cores=_NC, num_subcores=_NS)
    params = pltpu.CompilerParams(use_tc_tiling_on_sc=False,
                                  needs_layout_passes=False)
    gather = pl.kernel(
        _sc_gather_body,
        out_type=jax.ShapeDtypeStruct((_F, _E_PAD), jnp.float32),
        mesh=mesh,
        compiler_params=params,
        scratch_types=[
            pltpu.VMEM((2, _NSUB, _SUB), jnp.int32),
            pltpu.VMEM((2, _STG, _F), jnp.float32),
            pltpu.VMEM((2, _F, _STG), jnp.float32),
            pltpu.SemaphoreType.DMA,
            pltpu.SemaphoreType.DMA,
            pltpu.SemaphoreType.DMA,
        ],
    )
    scatter = pl.kernel(
        _sc_scatter_body,
        out_type=[
            jax.ShapeDtypeStruct((_NC, _F, _N_PAD), jnp.float32),
            jax.ShapeDtypeStruct((_NW, _N_PAD), jnp.float32),
        ],
        mesh=mesh,
        compiler_params=params,
        scratch_types=[
            pltpu.VMEM((2, _NSUB, _SUB), jnp.int32),
            pltpu.VMEM((2, _STG, _F), jnp.float32),
            pltpu.VMEM((2, _F, _STG), jnp.float32),
            pltpu.VMEM((_N_PAD,), jnp.float32),
            pltpu.VMEM((_STRIPE, _F), jnp.float32),
            pltpu.VMEM((_F, _STRIPE), jnp.float32),
            pltpu.VMEM_SHARED((_N_PAD, _F), jnp.float32),
            pltpu.SemaphoreType.DMA,
            pltpu.SemaphoreType.DMA,
        ],
    )
    return gather, scatter


_BLK = 16384


def _tc_msg_body(ea_ref, xj_ref, w1t_ref, b1c_ref, w2t_ref, b2c_ref,
                 w3pt_ref, b3pt_ref, msg_ref):
    f32 = jnp.float32
    h = jnp.dot(w1t_ref[...], ea_ref[...], preferred_element_type=f32)
    h = jnp.maximum(h + b1c_ref[...], 0.0)
    h = jnp.dot(w2t_ref[...], h, preferred_element_type=f32)
    h = jnp.maximum(h + b2c_ref[...], 0.0)
    tT = jnp.dot(w3pt_ref[...], h, preferred_element_type=f32) + b3pt_ref[...]
    xj = xj_ref[...]
    acc = tT[0:_F, :] * xj[0:1, :]
    for i in range(1, _F):
        acc = acc + tT[i * _F:(i + 1) * _F, :] * xj[i:i + 1, :]
    msg_ref[...] = acc


def _tc_msg(eaT, xjT, w1t, b1c, w2t, b2c, w3pt, b3pt):
    grid = _E_PAD // _BLK
    blk = lambda i: (jnp.int32(0), i)
    fixed = lambda i: (jnp.int32(0), jnp.int32(0))
    full = lambda shape: pl.BlockSpec(shape, fixed)
    return pl.pallas_call(
        _tc_msg_body,
        grid=(grid,),
        in_specs=[
            pl.BlockSpec((4, _BLK), blk),
            pl.BlockSpec((_F, _BLK), blk),
            full((_F, 4)), full((_F, 1)), full((32, _F)), full((32, 1)),
            full((256, 32)), full((256, 1)),
        ],
        out_specs=pl.BlockSpec((_F, _BLK), blk),
        out_shape=jax.ShapeDtypeStruct((_F, _E_PAD), jnp.float32),
        compiler_params=pltpu.CompilerParams(
            dimension_semantics=("arbitrary",)),
    )(eaT, xjT, w1t, b1c, w2t, b2c, w3pt, b3pt)


def _tc_final_body(agg_ref, deg_ref, gamma_ref, beta_ref, wf_ref, bf_ref,
                   out_ref):
    agg = agg_ref[0, :, :] + agg_ref[1, :, :]
    deg = jnp.sum(deg_ref[...], axis=0, keepdims=True)
    deg = jnp.maximum(deg, 1.0)
    out = agg / deg
    cid = lax.broadcasted_iota(jnp.int32, (_F, _N_PAD), 1)
    valid = cid < _N
    outm = jnp.where(valid, out, 0.0)
    inv_n = 1.0 / _N
    mu = jnp.sum(outm, axis=1, keepdims=True) * inv_n
    ex2 = jnp.sum(outm * outm, axis=1, keepdims=True) * inv_n
    var = ex2 - mu * mu
    scale = lax.rsqrt(var + 1e-5) * gamma_ref[...]
    out = (out - mu) * scale + beta_ref[...]
    out = jnp.maximum(out, 0.0)
    out_ref[...] = lax.dot_general(
        wf_ref[...], out, (((0,), (0,)), ((), ())),
        preferred_element_type=jnp.float32) + bf_ref[...]


def _tc_final(aggT, deg32, gamma, beta, wf, bf):
    return pl.pallas_call(
        _tc_final_body,
        out_shape=jax.ShapeDtypeStruct((_NOUT, _N_PAD), jnp.float32),
    )(aggT, deg32, gamma, beta, wf, bf)


def kernel(x, edge_index, edge_attr, W1, b1, W2, b2, W3, b3, gamma, beta,
           Wf, bf):
    f32 = jnp.float32
    x = x.astype(f32)
    src = edge_index[0].astype(jnp.int32)
    dst = edge_index[1].astype(jnp.int32)
    npad = _E_PAD - _E
    src = jnp.concatenate([src, jnp.zeros((npad,), jnp.int32)])
    dst = jnp.concatenate([dst, jnp.full((npad,), _TRASH, jnp.int32)])
    eaT = jnp.concatenate(
        [edge_attr.astype(f32).T, jnp.zeros((4, npad), f32)], axis=1)

    W3pt = W3.astype(f32).reshape(32, _F, _F).transpose(2, 1, 0).reshape(256, 32)
    b3pt = b3.astype(f32).reshape(_F, _F).T.reshape(256, 1)

    src2 = src.reshape(_E_PAD // _SUB, _SUB)
    dst2 = dst.reshape(_E_PAD // _SUB, _SUB)
    sc_gather, sc_scatter = _sc_kernels()
    xjT = sc_gather(x, src2)
    msgT = _tc_msg(eaT, xjT, W1.astype(f32).T, b1.astype(f32).reshape(_F, 1),
                   W2.astype(f32).T, b2.astype(f32).reshape(32, 1),
                   W3pt, b3pt)
    aggT, deg32 = sc_scatter(msgT, dst2)
    outT = _tc_final(aggT, deg32, gamma.astype(f32).reshape(_F, 1),
                     beta.astype(f32).reshape(_F, 1), Wf.astype(f32),
                     bf.astype(f32).reshape(_NOUT, 1))
    return outT[:, :_N].T.astype(jnp.float64)


# speedup vs baseline: 55.9354x; 1.0048x over previous
import functools

import jax
import jax.numpy as jnp
from jax import lax
from jax.experimental import pallas as pl
from jax.experimental.pallas import tpu as pltpu
from jax.experimental.pallas import tpu_sc as plsc

_N = 10000
_E = 160000
_F = 16
_NOUT = 40

_NC = 2
_NS = 16
_NW = _NC * _NS

_SUB = 128
_STG = 1280
_NSUB = _STG // _SUB
_NSTG = 4
_EPW = _STG * _NSTG
_E_PAD = _EPW * _NW

_STRIPE = 640
_N_PAD = _STRIPE * _NS
_TRASH = _N

_i32 = jnp.int32


def _iota16():
    return lax.iota(_i32, 16)


def _sc_gather_body(x_hbm, src2_hbm, xjT_hbm, idx2, rows2, strip2,
                    sem_i, sem_g, sem_o):
    c = lax.axis_index("c")
    s = lax.axis_index("s")
    wid = s * _i32(_NC) + c
    base = wid * _i32(_EPW)
    base_row = wid * _i32(_EPW // _SUB)

    def load_idx(g):
        return pltpu.async_copy(
            src2_hbm.at[pl.ds(base_row + _i32(g * _NSUB), _NSUB)],
            idx2.at[_i32(g & 1)], sem_i)

    def issue_gathers(g):
        b = g & 1
        return [pltpu.async_copy(
            x_hbm.at[idx2.at[_i32(b), _i32(j)]],
            rows2.at[_i32(b)].at[pl.ds(_i32(j * _SUB), _SUB)], sem_g)
            for j in range(_NSUB)]

    idx_d = load_idx(0)
    idx_d.wait()
    gd = {0: issue_gathers(0)}
    if _NSTG > 1:
        idx_d = load_idx(1)
    out_d = [None, None]
    for g in range(_NSTG):
        b = g & 1
        for d in gd.pop(g):
            d.wait()
        if g + 1 < _NSTG:
            idx_d.wait()
            gd[g + 1] = issue_gathers(g + 1)
            if g + 2 < _NSTG:
                idx_d = load_idx(g + 2)
        if out_d[b] is not None:
            for d in out_d[b]:
                d.wait()
        rows_b = rows2.at[_i32(b)]

        def tr_body(l8, carry):
            ridx = l8 * _i32(16) + _iota16()
            for f in range(_F):
                cidx = jnp.full((16,), f, _i32)
                v = plsc.load_gather(rows_b, [ridx, cidx])
                strip2[_i32(b), _i32(f), pl.ds(l8 * _i32(16), 16)] = v
            return carry

        lax.fori_loop(_i32(0), _i32(_STG // 16), tr_body, _i32(0))
        off = base + _i32(g * _STG)
        out_d[b] = [pltpu.async_copy(
            strip2.at[_i32(b), _i32(f)],
            xjT_hbm.at[_i32(f)].at[pl.ds(off, _STG)], sem_o)
            for f in range(_F)]
    for ds_ in out_d:
        if ds_ is not None:
            for d in ds_:
                d.wait()


def _sc_scatter_body(msgT_hbm, dst2_hbm, aggT_hbm, deg_hbm, idx2, rows2,
                     strip2, deg_v, zbuf_v, aggT_v, agg_sh, sem_i, sem_m):
    c = lax.axis_index("c")
    s = lax.axis_index("s")
    wid = s * _i32(_NC) + c
    base = wid * _i32(_EPW)
    base_row = wid * _i32(_EPW // _SUB)
    z16 = jnp.zeros((_F,), jnp.float32)

    def load_idx(g):
        return pltpu.async_copy(
            dst2_hbm.at[pl.ds(base_row + _i32(g * _NSUB), _NSUB)],
            idx2.at[_i32(g & 1)], sem_i)

    def load_msg(g):
        off = base + _i32(g * _STG)
        return [pltpu.async_copy(
            msgT_hbm.at[_i32(f)].at[pl.ds(off, _STG)],
            strip2.at[_i32(g & 1), _i32(f)], sem_m)
            for f in range(_F)]

    idx_d = load_idx(0)
    msg_d = load_msg(0)

    def zrow(i, carry):
        zbuf_v[i, :] = z16
        return carry

    lax.fori_loop(_i32(0), _i32(_STRIPE), zrow, _i32(0))

    def zdeg(i, carry):
        deg_v[pl.ds(i * _i32(_F), _F)] = z16
        return carry

    lax.fori_loop(_i32(0), _i32(_N_PAD // _F), zdeg, _i32(0))

    pltpu.sync_copy(zbuf_v, agg_sh.at[pl.ds(s * _i32(_STRIPE), _STRIPE)])
    plsc.subcore_barrier()

    ones16 = jnp.ones((_F,), jnp.float32)
    for g in range(_NSTG):
        b = g & 1
        idx_d.wait()
        for d in msg_d:
            d.wait()
        rows_b = rows2.at[_i32(b)]

        def tr_body(l8, carry):
            ridx = l8 * _i32(16) + _iota16()
            for f in range(_F):
                cidx = jnp.full((16,), f, _i32)
                v = strip2[_i32(b), _i32(f), pl.ds(l8 * _i32(16), 16)]
                plsc.store_scatter(rows_b, [ridx, cidx], v)
            return carry

        lax.fori_loop(_i32(0), _i32(_STG // 16), tr_body, _i32(0))
        if g + 1 < _NSTG:
            idx_d = load_idx(g + 1)
            msg_d = load_msg(g + 1)
        for j in range(_NSUB):
            for i in range(_SUB // _F):
                iv = idx2[_i32(b), _i32(j), pl.ds(_i32(i * _F), _F)]
                plsc.addupdate_scatter(deg_v, [iv], ones16)
        for j in range(_NSUB):
            pltpu.sync_copy(
                rows2.at[_i32(b)].at[pl.ds(_i32(j * _SUB), _SUB)],
                agg_sh.at[idx2.at[_i32(b), _i32(j)]], add=True)
    plsc.subcore_barrier()

    pltpu.sync_copy(agg_sh.at[pl.ds(s * _i32(_STRIPE), _STRIPE)], zbuf_v)

    def trs_body(l8, carry):
        ridx = l8 * _i32(16) + _iota16()
        for f in range(_F):
            cidx = jnp.full((16,), f, _i32)
            v = plsc.load_gather(zbuf_v, [ridx, cidx])
            aggT_v[_i32(f), pl.ds(l8 * _i32(16), 16)] = v
        return carry

    lax.fori_loop(_i32(0), _i32(_STRIPE // 16), trs_body, _i32(0))
    pltpu.sync_copy(aggT_v,
                    aggT_hbm.at[c].at[:, pl.ds(s * _i32(_STRIPE), _STRIPE)])
    pltpu.sync_copy(deg_v, deg_hbm.at[wid])


@functools.cache
def _sc_kernels():
    mesh = plsc.VectorSubcoreMesh(core_axis_name="c", subcore_axis_name="s",
                                  num_cores=_NC, num_subcores=_NS)
    params = pltpu.CompilerParams(use_tc_tiling_on_sc=False,
                                  needs_layout_passes=False)
    gather = pl.kernel(
        _sc_gather_body,
        out_type=jax.ShapeDtypeStruct((_F, _E_PAD), jnp.float32),
        mesh=mesh,
        compiler_params=params,
        scratch_types=[
            pltpu.VMEM((2, _NSUB, _SUB), jnp.int32),
            pltpu.VMEM((2, _STG, _F), jnp.float32),
            pltpu.VMEM((2, _F, _STG), jnp.float32),
            pltpu.SemaphoreType.DMA,
            pltpu.SemaphoreType.DMA,
            pltpu.SemaphoreType.DMA,
        ],
    )
    scatter = pl.kernel(
        _sc_scatter_body,
        out_type=[
            jax.ShapeDtypeStruct((_NC, _F, _N_PAD), jnp.float32),
            jax.ShapeDtypeStruct((_NW, _N_PAD), jnp.float32),
        ],
        mesh=mesh,
        compiler_params=params,
        scratch_types=[
            pltpu.VMEM((2, _NSUB, _SUB), jnp.int32),
            pltpu.VMEM((2, _STG, _F), jnp.float32),
            pltpu.VMEM((2, _F, _STG), jnp.float32),
            pltpu.VMEM((_N_PAD,), jnp.float32),
            pltpu.VMEM((_STRIPE, _F), jnp.float32),
            pltpu.VMEM((_F, _STRIPE), jnp.float32),
            pltpu.VMEM_SHARED((_N_PAD, _F), jnp.float32),
            pltpu.SemaphoreType.DMA,
            pltpu.SemaphoreType.DMA,
        ],
    )
    return gather, scatter


_BLK = 16384


def _tc_msg_body(ea_ref, xj_ref, w1t_ref, b1c_ref, w2t_ref, b2c_ref,
                 w3pt_ref, b3pt_ref, msg_ref):
    f32 = jnp.float32
    h = jnp.dot(w1t_ref[...], ea_ref[...], preferred_element_type=f32)
    h = jnp.maximum(h + b1c_ref[...], 0.0)
    h = jnp.dot(w2t_ref[...], h, preferred_element_type=f32)
    h = jnp.maximum(h + b2c_ref[...], 0.0)
    tT = jnp.dot(w3pt_ref[...], h, preferred_element_type=f32) + b3pt_ref[...]
    xj = xj_ref[...]
    acc = tT[0:_F, :] * xj[0:1, :]
    for i in range(1, _F):
        acc = acc + tT[i * _F:(i + 1) * _F, :] * xj[i:i + 1, :]
    msg_ref[...] = acc


def _tc_msg(eaT, xjT, w1t, b1c, w2t, b2c, w3pt, b3pt):
    grid = _E_PAD // _BLK
    blk = lambda i: (jnp.int32(0), i)
    fixed = lambda i: (jnp.int32(0), jnp.int32(0))
    full = lambda shape: pl.BlockSpec(shape, fixed)
    return pl.pallas_call(
        _tc_msg_body,
        grid=(grid,),
        in_specs=[
            pl.BlockSpec((4, _BLK), blk),
            pl.BlockSpec((_F, _BLK), blk),
            full((_F, 4)), full((_F, 1)), full((32, _F)), full((32, 1)),
            full((256, 32)), full((256, 1)),
        ],
        out_specs=pl.BlockSpec((_F, _BLK), blk),
        out_shape=jax.ShapeDtypeStruct((_F, _E_PAD), jnp.float32),
        compiler_params=pltpu.CompilerParams(
            dimension_semantics=("arbitrary",)),
    )(eaT, xjT, w1t, b1c, w2t, b2c, w3pt, b3pt)


def _tc_final_body(agg_ref, deg_ref, gamma_ref, beta_ref, wf_ref, bf_ref,
                   out_ref):
    agg = agg_ref[0, :, :] + agg_ref[1, :, :]
    deg = jnp.sum(deg_ref[...], axis=0, keepdims=True)
    deg = jnp.maximum(deg, 1.0)
    out = agg / deg
    cid = lax.broadcasted_iota(jnp.int32, (_F, _N_PAD), 1)
    valid = cid < _N
    outm = jnp.where(valid, out, 0.0)
    inv_n = 1.0 / _N
    mu = jnp.sum(outm, axis=1, keepdims=True) * inv_n
    ex2 = jnp.sum(outm * outm, axis=1, keepdims=True) * inv_n
    var = ex2 - mu * mu
    scale = lax.rsqrt(var + 1e-5) * gamma_ref[...]
    out = (out - mu) * scale + beta_ref[...]
    out = jnp.maximum(out, 0.0)
    out_ref[...] = lax.dot_general(
        wf_ref[...], out, (((0,), (0,)), ((), ())),
        preferred_element_type=jnp.float32) + bf_ref[...]


def _tc_final(aggT, deg32, gamma, beta, wf, bf):
    return pl.pallas_call(
        _tc_final_body,
        out_shape=jax.ShapeDtypeStruct((_NOUT, _N_PAD), jnp.float32),
    )(aggT, deg32, gamma, beta, wf, bf)


def kernel(x, edge_index, edge_attr, W1, b1, W2, b2, W3, b3, gamma, beta,
           Wf, bf):
    f32 = jnp.float32
    x = x.astype(f32)
    src = edge_index[0].astype(jnp.int32)
    dst = edge_index[1].astype(jnp.int32)
    npad = _E_PAD - _E
    src = jnp.concatenate([src, jnp.zeros((npad,), jnp.int32)])
    dst = jnp.concatenate([dst, jnp.full((npad,), _TRASH, jnp.int32)])
    eaT = jnp.concatenate(
        [edge_attr.astype(f32).T, jnp.zeros((4, npad), f32)], axis=1)

    W3pt = W3.astype(f32).reshape(32, _F, _F).transpose(2, 1, 0).reshape(256, 32)
    b3pt = b3.astype(f32).reshape(_F, _F).T.reshape(256, 1)

    src2 = src.reshape(_E_PAD // _SUB, _SUB)
    dst2 = dst.reshape(_E_PAD // _SUB, _SUB)
    sc_gather, sc_scatter = _sc_kernels()
    xjT = sc_gather(x, src2)
    msgT = _tc_msg(eaT, xjT, W1.astype(f32).T, b1.astype(f32).reshape(_F, 1),
                   W2.astype(f32).T, b2.astype(f32).reshape(32, 1),
                   W3pt, b3pt)
    aggT, deg32 = sc_scatter(msgT, dst2)
    outT = _tc_final(aggT, deg32, gamma.astype(f32).reshape(_F, 1),
                     beta.astype(f32).reshape(_F, 1), Wf.astype(f32),
                     bf.astype(f32).reshape(_NOUT, 1))
    return outT[:, :_N].T.astype(jnp.float64)


# speedup vs baseline: 62.3325x; 1.1144x over previous
import functools

import jax
import jax.numpy as jnp
from jax import lax
from jax.experimental import pallas as pl
from jax.experimental.pallas import tpu as pltpu
from jax.experimental.pallas import tpu_sc as plsc

_N = 10000
_E = 160000
_F = 16
_NOUT = 40

_NC = 2
_NS = 16
_NW = _NC * _NS

_SUB = 128
_STG = 1280
_NSUB = _STG // _SUB
_NSTG = 2
_EPW = _STG * _NSTG
_E_HALF = _EPW * _NW
_E_PAD = 2 * _E_HALF

_STRIPE = 640
_N_PAD = _STRIPE * _NS
_TRASH = _N

_i32 = jnp.int32


def _iota16():
    return lax.iota(_i32, 16)


def _sc_gather_body(x_hbm, src2_hbm, xjT_hbm, idx2, rows2, strip2,
                    sem_i, sem_g, sem_o):
    c = lax.axis_index("c")
    s = lax.axis_index("s")
    wid = s * _i32(_NC) + c
    base = wid * _i32(_EPW)
    base_row = wid * _i32(_EPW // _SUB)

    def load_idx(g):
        return pltpu.async_copy(
            src2_hbm.at[pl.ds(base_row + _i32(g * _NSUB), _NSUB)],
            idx2.at[_i32(g & 1)], sem_i)

    def issue_gathers(g):
        b = g & 1
        return [pltpu.async_copy(
            x_hbm.at[idx2.at[_i32(b), _i32(j)]],
            rows2.at[_i32(b)].at[pl.ds(_i32(j * _SUB), _SUB)], sem_g)
            for j in range(_NSUB)]

    idx_d = load_idx(0)
    idx_d.wait()
    gd = {0: issue_gathers(0)}
    if _NSTG > 1:
        idx_d = load_idx(1)
    out_d = [None, None]
    for g in range(_NSTG):
        b = g & 1
        for d in gd.pop(g):
            d.wait()
        if g + 1 < _NSTG:
            idx_d.wait()
            gd[g + 1] = issue_gathers(g + 1)
            if g + 2 < _NSTG:
                idx_d = load_idx(g + 2)
        if out_d[b] is not None:
            for d in out_d[b]:
                d.wait()
        rows_b = rows2.at[_i32(b)]

        def tr_body(l8, carry):
            ridx = l8 * _i32(16) + _iota16()
            for f in range(_F):
                cidx = jnp.full((16,), f, _i32)
                v = plsc.load_gather(rows_b, [ridx, cidx])
                strip2[_i32(b), _i32(f), pl.ds(l8 * _i32(16), 16)] = v
            return carry

        lax.fori_loop(_i32(0), _i32(_STG // 16), tr_body, _i32(0))
        off = base + _i32(g * _STG)
        out_d[b] = [pltpu.async_copy(
            strip2.at[_i32(b), _i32(f)],
            xjT_hbm.at[_i32(f)].at[pl.ds(off, _STG)], sem_o)
            for f in range(_F)]
    for ds_ in out_d:
        if ds_ is not None:
            for d in ds_:
                d.wait()


def _sc_scatter_body(msgT_hbm, dst2_hbm, aggT_hbm, deg_hbm, idx2, rows2,
                     strip2, deg_v, zbuf_v, aggT_v, agg_sh, sem_i, sem_m):
    c = lax.axis_index("c")
    s = lax.axis_index("s")
    wid = s * _i32(_NC) + c
    base = wid * _i32(_EPW)
    base_row = wid * _i32(_EPW // _SUB)
    z16 = jnp.zeros((_F,), jnp.float32)

    def load_idx(g):
        return pltpu.async_copy(
            dst2_hbm.at[pl.ds(base_row + _i32(g * _NSUB), _NSUB)],
            idx2.at[_i32(g & 1)], sem_i)

    def load_msg(g):
        off = base + _i32(g * _STG)
        return [pltpu.async_copy(
            msgT_hbm.at[_i32(f)].at[pl.ds(off, _STG)],
            strip2.at[_i32(g & 1), _i32(f)], sem_m)
            for f in range(_F)]

    idx_d = load_idx(0)
    msg_d = load_msg(0)

    def zrow(i, carry):
        zbuf_v[i, :] = z16
        return carry

    lax.fori_loop(_i32(0), _i32(_STRIPE), zrow, _i32(0))

    def zdeg(i, carry):
        deg_v[pl.ds(i * _i32(_F), _F)] = z16
        return carry

    lax.fori_loop(_i32(0), _i32(_N_PAD // _F), zdeg, _i32(0))

    pltpu.sync_copy(zbuf_v, agg_sh.at[pl.ds(s * _i32(_STRIPE), _STRIPE)])
    plsc.subcore_barrier()

    ones16 = jnp.ones((_F,), jnp.float32)
    for g in range(_NSTG):
        b = g & 1
        idx_d.wait()
        for d in msg_d:
            d.wait()
        rows_b = rows2.at[_i32(b)]

        def tr_body(l8, carry):
            ridx = l8 * _i32(16) + _iota16()
            for f in range(_F):
                cidx = jnp.full((16,), f, _i32)
                v = strip2[_i32(b), _i32(f), pl.ds(l8 * _i32(16), 16)]
                plsc.store_scatter(rows_b, [ridx, cidx], v)
            return carry

        lax.fori_loop(_i32(0), _i32(_STG // 16), tr_body, _i32(0))
        if g + 1 < _NSTG:
            idx_d = load_idx(g + 1)
            msg_d = load_msg(g + 1)
        for j in range(_NSUB):
            for i in range(_SUB // _F):
                iv = idx2[_i32(b), _i32(j), pl.ds(_i32(i * _F), _F)]
                plsc.addupdate_scatter(deg_v, [iv], ones16)
        for j in range(_NSUB):
            pltpu.sync_copy(
                rows2.at[_i32(b)].at[pl.ds(_i32(j * _SUB), _SUB)],
                agg_sh.at[idx2.at[_i32(b), _i32(j)]], add=True)
    plsc.subcore_barrier()

    pltpu.sync_copy(agg_sh.at[pl.ds(s * _i32(_STRIPE), _STRIPE)], zbuf_v)

    def trs_body(l8, carry):
        ridx = l8 * _i32(16) + _iota16()
        for f in range(_F):
            cidx = jnp.full((16,), f, _i32)
            v = plsc.load_gather(zbuf_v, [ridx, cidx])
            aggT_v[_i32(f), pl.ds(l8 * _i32(16), 16)] = v
        return carry

    lax.fori_loop(_i32(0), _i32(_STRIPE // 16), trs_body, _i32(0))
    pltpu.sync_copy(aggT_v,
                    aggT_hbm.at[c].at[:, pl.ds(s * _i32(_STRIPE), _STRIPE)])
    pltpu.sync_copy(deg_v, deg_hbm.at[wid])


@functools.cache
def _sc_kernels():
    mesh = plsc.VectorSubcoreMesh(core_axis_name="c", subcore_axis_name="s",
                                  num_cores=_NC, num_subcores=_NS)
    params = pltpu.CompilerParams(use_tc_tiling_on_sc=False,
                                  needs_layout_passes=False)
    gather = pl.kernel(
        _sc_gather_body,
        out_type=jax.ShapeDtypeStruct((_F, _E_HALF), jnp.float32),
        mesh=mesh,
        compiler_params=params,
        scratch_types=[
            pltpu.VMEM((2, _NSUB, _SUB), jnp.int32),
            pltpu.VMEM((2, _STG, _F), jnp.float32),
            pltpu.VMEM((2, _F, _STG), jnp.float32),
            pltpu.SemaphoreType.DMA,
            pltpu.SemaphoreType.DMA,
            pltpu.SemaphoreType.DMA,
        ],
    )
    scatter = pl.kernel(
        _sc_scatter_body,
        out_type=[
            jax.ShapeDtypeStruct((_NC, _F, _N_PAD), jnp.float32),
            jax.ShapeDtypeStruct((_NW, _N_PAD), jnp.float32),
        ],
        mesh=mesh,
        compiler_params=params,
        scratch_types=[
            pltpu.VMEM((2, _NSUB, _SUB), jnp.int32),
            pltpu.VMEM((2, _STG, _F), jnp.float32),
            pltpu.VMEM((2, _F, _STG), jnp.float32),
            pltpu.VMEM((_N_PAD,), jnp.float32),
            pltpu.VMEM((_STRIPE, _F), jnp.float32),
            pltpu.VMEM((_F, _STRIPE), jnp.float32),
            pltpu.VMEM_SHARED((_N_PAD, _F), jnp.float32),
            pltpu.SemaphoreType.DMA,
            pltpu.SemaphoreType.DMA,
        ],
    )
    return gather, scatter


_BLK = 16384


def _tc_msg_body(ea_ref, xj_ref, w1t_ref, b1c_ref, w2t_ref, b2c_ref,
                 w3pt_ref, b3pt_ref, msg_ref):
    f32 = jnp.float32
    h = jnp.dot(w1t_ref[...], ea_ref[...], preferred_element_type=f32)
    h = jnp.maximum(h + b1c_ref[...], 0.0)
    h = jnp.dot(w2t_ref[...], h, preferred_element_type=f32)
    h = jnp.maximum(h + b2c_ref[...], 0.0)
    tT = jnp.dot(w3pt_ref[...], h, preferred_element_type=f32) + b3pt_ref[...]
    xj = xj_ref[...]
    acc = tT[0:_F, :] * xj[0:1, :]
    for i in range(1, _F):
        acc = acc + tT[i * _F:(i + 1) * _F, :] * xj[i:i + 1, :]
    msg_ref[...] = acc


def _tc_msg(eaT, xjT, w1t, b1c, w2t, b2c, w3pt, b3pt):
    grid = _E_HALF // _BLK
    blk = lambda i: (jnp.int32(0), i)
    fixed = lambda i: (jnp.int32(0), jnp.int32(0))
    full = lambda shape: pl.BlockSpec(shape, fixed)
    return pl.pallas_call(
        _tc_msg_body,
        grid=(grid,),
        in_specs=[
            pl.BlockSpec((4, _BLK), blk),
            pl.BlockSpec((_F, _BLK), blk),
            full((_F, 4)), full((_F, 1)), full((32, _F)), full((32, 1)),
            full((256, 32)), full((256, 1)),
        ],
        out_specs=pl.BlockSpec((_F, _BLK), blk),
        out_shape=jax.ShapeDtypeStruct((_F, _E_HALF), jnp.float32),
        compiler_params=pltpu.CompilerParams(
            dimension_semantics=("arbitrary",)),
    )(eaT, xjT, w1t, b1c, w2t, b2c, w3pt, b3pt)


def _tc_final_body(agg_ref, agg1_ref, deg_ref, deg1_ref, gamma_ref, beta_ref,
                   wf_ref, bf_ref, out_ref):
    agg = (agg_ref[0, :, :] + agg_ref[1, :, :] +
           agg1_ref[0, :, :] + agg1_ref[1, :, :])
    deg = (jnp.sum(deg_ref[...], axis=0, keepdims=True) +
           jnp.sum(deg1_ref[...], axis=0, keepdims=True))
    deg = jnp.maximum(deg, 1.0)
    out = agg / deg
    cid = lax.broadcasted_iota(jnp.int32, (_F, _N_PAD), 1)
    valid = cid < _N
    outm = jnp.where(valid, out, 0.0)
    inv_n = 1.0 / _N
    mu = jnp.sum(outm, axis=1, keepdims=True) * inv_n
    ex2 = jnp.sum(outm * outm, axis=1, keepdims=True) * inv_n
    var = ex2 - mu * mu
    scale = lax.rsqrt(var + 1e-5) * gamma_ref[...]
    out = (out - mu) * scale + beta_ref[...]
    out = jnp.maximum(out, 0.0)
    out_ref[...] = lax.dot_general(
        wf_ref[...], out, (((0,), (0,)), ((), ())),
        preferred_element_type=jnp.float32) + bf_ref[...]


def _tc_final(aggT0, aggT1, deg0, deg1, gamma, beta, wf, bf):
    return pl.pallas_call(
        _tc_final_body,
        out_shape=jax.ShapeDtypeStruct((_NOUT, _N_PAD), jnp.float32),
    )(aggT0, aggT1, deg0, deg1, gamma, beta, wf, bf)


def kernel(x, edge_index, edge_attr, W1, b1, W2, b2, W3, b3, gamma, beta,
           Wf, bf):
    f32 = jnp.float32
    x = x.astype(f32)
    src = edge_index[0].astype(jnp.int32)
    dst = edge_index[1].astype(jnp.int32)
    npad = _E_PAD - _E
    src = jnp.concatenate([src, jnp.zeros((npad,), jnp.int32)])
    dst = jnp.concatenate([dst, jnp.full((npad,), _TRASH, jnp.int32)])
    eaT = jnp.concatenate(
        [edge_attr.astype(f32).T, jnp.zeros((4, npad), f32)], axis=1)

    W3pt = W3.astype(f32).reshape(32, _F, _F).transpose(2, 1, 0).reshape(256, 32)
    b3pt = b3.astype(f32).reshape(_F, _F).T.reshape(256, 1)

    src2 = src.reshape(_E_PAD // _SUB, _SUB)
    dst2 = dst.reshape(_E_PAD // _SUB, _SUB)
    hrows = _E_HALF // _SUB
    sc_gather, sc_scatter = _sc_kernels()
    w1t = W1.astype(f32).T
    b1c = b1.astype(f32).reshape(_F, 1)
    w2t = W2.astype(f32).T
    b2c = b2.astype(f32).reshape(32, 1)
    xjT0 = sc_gather(x, src2[:hrows])
    xjT1 = sc_gather(x, src2[hrows:])
    msgT0 = _tc_msg(eaT[:, :_E_HALF], xjT0, w1t, b1c, w2t, b2c, W3pt, b3pt)
    msgT1 = _tc_msg(eaT[:, _E_HALF:], xjT1, w1t, b1c, w2t, b2c, W3pt, b3pt)
    aggT0, deg0 = sc_scatter(msgT0, dst2[:hrows])
    aggT1, deg1 = sc_scatter(msgT1, dst2[hrows:])
    outT = _tc_final(aggT0, aggT1, deg0, deg1, gamma.astype(f32).reshape(_F, 1),
                     beta.astype(f32).reshape(_F, 1), Wf.astype(f32),
                     bf.astype(f32).reshape(_NOUT, 1))
    return outT[:, :_N].T.astype(jnp.float64)
